# baseline (speedup 1.0000x reference)
import functools

import jax
import jax.numpy as jnp
from jax import lax
from jax.experimental import pallas as pl
from jax.experimental.pallas import tpu as pltpu
from jax.experimental.pallas import tpu_sc as plsc

N_NODES = 20000
N_EDGES = 100000
D = 512
C_NODE = 151
C_EDGE = 51

_NW = 32
_PER_W = N_EDGES * D // _NW
_CHUNK = 100000
_NCHUNK = _PER_W // _CHUNK
_DEPTH = 3


def _sc_zero_fill(zsrc):
    mesh = plsc.VectorSubcoreMesh(core_axis_name="c", subcore_axis_name="s")

    @functools.partial(
        pl.kernel, mesh=mesh,
        out_type=jax.ShapeDtypeStruct((N_EDGES * D,), jnp.float32),
        scratch_types=[
            pltpu.VMEM((_CHUNK,), jnp.float32),
            pltpu.SemaphoreType.DMA,
        ],
    )
    def zero_fill(zsrc_hbm, out_hbm, zbuf, sem):
        wid = lax.axis_index("s") * 2 + lax.axis_index("c")
        base = wid * _PER_W
        pltpu.sync_copy(zsrc_hbm, zbuf)

        def body(k, carry):
            pltpu.async_copy(
                zbuf, out_hbm.at[pl.ds(base + k * _CHUNK, _CHUNK)], sem)

            @pl.when(k >= _DEPTH)
            def _():
                pltpu.make_async_copy(
                    zbuf, out_hbm.at[pl.ds(base, _CHUNK)], sem).wait()

            return carry

        lax.fori_loop(0, _NCHUNK, body, 0)

        def drain(k, carry):
            pltpu.make_async_copy(
                zbuf, out_hbm.at[pl.ds(base, _CHUNK)], sem).wait()
            return carry

        lax.fori_loop(0, _DEPTH, drain, 0)

    return zero_fill(zsrc)

BLK_E = 4096
BLK_N = 4096


def _edge_block(x_ref, w_ref, raw_ref):
    raw_ref[...] = jax.lax.dot_general(
        w_ref[...], x_ref[...], (((0,), (1,)), ((), ())),
        preferred_element_type=jnp.float32)


def _node_block(x_ref, w_ref, wt_ref, raw_ref, h_ref):
    raw_t = jax.lax.dot_general(
        w_ref[...], x_ref[...], (((0,), (1,)), ((), ())),
        preferred_element_type=jnp.float32)
    raw_ref[...] = raw_t
    m = jnp.max(raw_t, axis=0, keepdims=True)
    e = jnp.exp(raw_t - m)
    att_t = e / jnp.sum(e, axis=0, keepdims=True)
    h_ref[...] = jax.lax.dot_general(
        att_t, wt_ref[...], (((0,), (0,)), ((), ())),
        preferred_element_type=jnp.float32)


def kernel(node_emb, edge_emb, is_training, gt_node_dists, gt_edge_dists,
           mode, edges_schema, nodes_schema):
    h_edge_emb = _sc_zero_fill(
        jnp.zeros((_CHUNK,), jnp.float32)).reshape(N_EDGES, D)
    raw_edge_t = pl.pallas_call(
        _edge_block,
        grid=(pl.cdiv(N_EDGES, BLK_E),),
        in_specs=[
            pl.BlockSpec((BLK_E, D), lambda i: (i, 0)),
            pl.BlockSpec((D, C_EDGE), lambda i: (0, 0)),
        ],
        out_specs=pl.BlockSpec((C_EDGE, BLK_E), lambda i: (0, i)),
        out_shape=jax.ShapeDtypeStruct((C_EDGE, N_EDGES), jnp.float32),
    )(edge_emb, edges_schema)
    raw_edge_class = raw_edge_t.T

    nodes_schema_t = jnp.swapaxes(nodes_schema, 0, 1)
    raw_node_t, h_node_emb = pl.pallas_call(
        _node_block,
        grid=(pl.cdiv(N_NODES, BLK_N),),
        in_specs=[
            pl.BlockSpec((BLK_N, D), lambda i: (i, 0)),
            pl.BlockSpec((D, C_NODE), lambda i: (0, 0)),
            pl.BlockSpec((C_NODE, D), lambda i: (0, 0)),
        ],
        out_specs=[
            pl.BlockSpec((C_NODE, BLK_N), lambda i: (0, i)),
            pl.BlockSpec((BLK_N, D), lambda i: (i, 0)),
        ],
        out_shape=[
            jax.ShapeDtypeStruct((C_NODE, N_NODES), jnp.float32),
            jax.ShapeDtypeStruct((N_NODES, D), jnp.float32),
        ],
    )(node_emb, nodes_schema, nodes_schema_t)
    raw_node_class = raw_node_t.T

    return (raw_edge_class, h_edge_emb, raw_node_class, h_node_emb)


# speedup vs baseline: 2.3591x; 2.3591x over previous
import jax
import jax.numpy as jnp
from jax.experimental import pallas as pl

N_NODES = 20000
N_EDGES = 100000
D = 512
C_NODE = 151
C_EDGE = 51

BLK_E = 4096
BLK_N = 2048


def _edge_block(x_ref, w_ref, raw_ref):
    raw_ref[...] = jax.lax.dot_general(
        w_ref[...], x_ref[...], (((0,), (1,)), ((), ())),
        preferred_element_type=jnp.float32)


def _node_block(x_ref, w_ref, wt_ref, raw_ref, h_ref):
    raw_t = jax.lax.dot_general(
        w_ref[...], x_ref[...], (((0,), (1,)), ((), ())),
        preferred_element_type=jnp.float32)
    raw_ref[...] = raw_t
    m = jnp.max(raw_t, axis=0, keepdims=True)
    e = jnp.exp(raw_t - m)
    att_t = e / jnp.sum(e, axis=0, keepdims=True)
    h_ref[...] = jax.lax.dot_general(
        att_t, wt_ref[...], (((0,), (0,)), ((), ())),
        preferred_element_type=jnp.float32)


def kernel(node_emb, edge_emb, is_training, gt_node_dists, gt_edge_dists,
           mode, edges_schema, nodes_schema):
    raw_edge_t = pl.pallas_call(
        _edge_block,
        grid=(pl.cdiv(N_EDGES, BLK_E),),
        in_specs=[
            pl.BlockSpec((BLK_E, D), lambda i: (i, 0)),
            pl.BlockSpec((D, C_EDGE), lambda i: (0, 0)),
        ],
        out_specs=pl.BlockSpec((C_EDGE, BLK_E), lambda i: (0, i)),
        out_shape=jax.ShapeDtypeStruct((C_EDGE, N_EDGES), jnp.float32),
    )(edge_emb, edges_schema)
    raw_edge_class = raw_edge_t.T
    h_edge_emb = jnp.zeros((N_EDGES, D), dtype=jnp.float32)

    nodes_schema_t = jnp.swapaxes(nodes_schema, 0, 1)
    raw_node_t, h_node_emb = pl.pallas_call(
        _node_block,
        grid=(pl.cdiv(N_NODES, BLK_N),),
        in_specs=[
            pl.BlockSpec((BLK_N, D), lambda i: (i, 0)),
            pl.BlockSpec((D, C_NODE), lambda i: (0, 0)),
            pl.BlockSpec((C_NODE, D), lambda i: (0, 0)),
        ],
        out_specs=[
            pl.BlockSpec((C_NODE, BLK_N), lambda i: (0, i)),
            pl.BlockSpec((BLK_N, D), lambda i: (i, 0)),
        ],
        out_shape=[
            jax.ShapeDtypeStruct((C_NODE, N_NODES), jnp.float32),
            jax.ShapeDtypeStruct((N_NODES, D), jnp.float32),
        ],
    )(node_emb, nodes_schema, nodes_schema_t)
    raw_node_class = raw_node_t.T

    return (raw_edge_class, h_edge_emb, raw_node_class, h_node_emb)


# speedup vs baseline: 2.3875x; 1.0121x over previous
import jax
import jax.numpy as jnp
from jax.experimental import pallas as pl

N_NODES = 20000
N_EDGES = 100000
D = 512
C_NODE = 151
C_EDGE = 51

BLK_E = 4096
BLK_N = 4096


def _edge_block(x_ref, w_ref, raw_ref):
    raw_ref[...] = jax.lax.dot_general(
        w_ref[...], x_ref[...], (((0,), (1,)), ((), ())),
        preferred_element_type=jnp.float32)


def _node_block(x_ref, w_ref, wt_ref, raw_ref, h_ref):
    raw_t = jax.lax.dot_general(
        w_ref[...], x_ref[...], (((0,), (1,)), ((), ())),
        preferred_element_type=jnp.float32)
    raw_ref[...] = raw_t
    m = jnp.max(raw_t, axis=0, keepdims=True)
    e = jnp.exp(raw_t - m)
    att_t = e / jnp.sum(e, axis=0, keepdims=True)
    h_ref[...] = jax.lax.dot_general(
        att_t, wt_ref[...], (((0,), (0,)), ((), ())),
        preferred_element_type=jnp.float32)


def kernel(node_emb, edge_emb, is_training, gt_node_dists, gt_edge_dists,
           mode, edges_schema, nodes_schema):
    raw_edge_t = pl.pallas_call(
        _edge_block,
        grid=(pl.cdiv(N_EDGES, BLK_E),),
        in_specs=[
            pl.BlockSpec((BLK_E, D), lambda i: (i, 0)),
            pl.BlockSpec((D, C_EDGE), lambda i: (0, 0)),
        ],
        out_specs=pl.BlockSpec((C_EDGE, BLK_E), lambda i: (0, i)),
        out_shape=jax.ShapeDtypeStruct((C_EDGE, N_EDGES), jnp.float32),
    )(edge_emb, edges_schema)
    raw_edge_class = raw_edge_t.T
    h_edge_emb = jnp.zeros((N_EDGES, D), dtype=jnp.float32)

    nodes_schema_t = jnp.swapaxes(nodes_schema, 0, 1)
    raw_node_t, h_node_emb = pl.pallas_call(
        _node_block,
        grid=(pl.cdiv(N_NODES, BLK_N),),
        in_specs=[
            pl.BlockSpec((BLK_N, D), lambda i: (i, 0)),
            pl.BlockSpec((D, C_NODE), lambda i: (0, 0)),
            pl.BlockSpec((C_NODE, D), lambda i: (0, 0)),
        ],
        out_specs=[
            pl.BlockSpec((C_NODE, BLK_N), lambda i: (0, i)),
            pl.BlockSpec((BLK_N, D), lambda i: (i, 0)),
        ],
        out_shape=[
            jax.ShapeDtypeStruct((C_NODE, N_NODES), jnp.float32),
            jax.ShapeDtypeStruct((N_NODES, D), jnp.float32),
        ],
    )(node_emb, nodes_schema, nodes_schema_t)
    raw_node_class = raw_node_t.T

    return (raw_edge_class, h_edge_emb, raw_node_class, h_node_emb)
